# Initial kernel scaffold; baseline (speedup 1.0000x reference)
import functools

import jax
import jax.numpy as jnp
from jax import lax
from jax.experimental import pallas as pl
from jax.experimental.pallas import tpu as pltpu
from jax.experimental.pallas import tpu_sc as plsc

N_NODES = 10000
N_EDGES = 320000
D_FEAT = 128
D_EDGE = 16
HIDDEN = 128

NC = 2
NS = 16
LANES = 16
NW = NC * NS
E_PER_W = N_EDGES // NW
BLK = 80
NBLK = E_PER_W // BLK
ROWS_PER_SUBCORE = N_NODES // NS
ZROWS = 125


def _proj_body(h_ref, ws_ref, wt_ref, hs_ref, ht_ref):
    h = h_ref[...]
    hs_ref[...] = jnp.dot(h, ws_ref[...], preferred_element_type=jnp.float32)
    ht_ref[...] = jnp.dot(h, wt_ref[...], preferred_element_type=jnp.float32)


def _proj(h, ws, wt):
    return pl.pallas_call(
        _proj_body,
        out_shape=[
            jax.ShapeDtypeStruct((N_NODES, HIDDEN), jnp.float32),
            jax.ShapeDtypeStruct((N_NODES, HIDDEN), jnp.float32),
        ],
    )(h, ws, wt)


_EA_BLK = 8000


def _ea_body(a_ref, wa_ref, b_ref, o_ref):
    o_ref[...] = (
        jnp.dot(a_ref[...], wa_ref[...], preferred_element_type=jnp.float32)
        + b_ref[...]
    )


def _ea(edge_attr, wa, b_edge):
    return pl.pallas_call(
        _ea_body,
        grid=(N_EDGES // _EA_BLK,),
        in_specs=[
            pl.BlockSpec((_EA_BLK, D_EDGE), lambda i: (i, 0)),
            pl.BlockSpec((D_EDGE, HIDDEN), lambda i: (0, 0)),
            pl.BlockSpec((1, HIDDEN), lambda i: (0, 0)),
        ],
        out_specs=pl.BlockSpec((_EA_BLK, HIDDEN), lambda i: (i, 0)),
        out_shape=jax.ShapeDtypeStruct((N_EDGES, HIDDEN), jnp.float32),
    )(edge_attr, wa, b_edge.reshape(1, HIDDEN))


def _sc_edge_body(hs_hbm, ht_hbm, ea_hbm, row_hbm, col_hbm, out_hbm,
                  rowv, colv, hsb, htb, eab, zbuf, agg, sem1, sem2, sem3):
    c = lax.axis_index("c")
    s = lax.axis_index("s")
    wid = c * NS + s
    base = wid * E_PER_W

    @pl.loop(0, ZROWS)
    def _zero_rows(i):
        @pl.loop(0, HIDDEN, step=LANES)
        def _zero_lanes(j):
            zbuf[i, pl.ds(j, LANES)] = jnp.zeros((LANES,), jnp.float32)

    @pl.loop(0, ROWS_PER_SUBCORE, step=ZROWS)
    def _zero_copy(r):
        pltpu.sync_copy(zbuf, agg.at[pl.ds(s * ROWS_PER_SUBCORE + r, ZROWS)])

    plsc.subcore_barrier()

    @pl.loop(0, NBLK)
    def _block(b):
        off = base + b * BLK
        pltpu.sync_copy(row_hbm.at[pl.ds(off, BLK)], rowv)
        pltpu.sync_copy(col_hbm.at[pl.ds(off, BLK)], colv)
        ga = pltpu.async_copy(hs_hbm.at[rowv], hsb, sem1)
        gb = pltpu.async_copy(ht_hbm.at[colv], htb, sem2)
        ge = pltpu.async_copy(ea_hbm.at[pl.ds(off, BLK)], eab, sem3)
        ga.wait()
        gb.wait()
        ge.wait()

        @pl.loop(0, BLK)
        def _edge(i):
            @pl.loop(0, HIDDEN, step=LANES)
            def _lanes(j):
                v = (hsb[i, pl.ds(j, LANES)]
                     + htb[i, pl.ds(j, LANES)]
                     + eab[i, pl.ds(j, LANES)])
                hsb[i, pl.ds(j, LANES)] = jnp.maximum(v, 0.0)

        pltpu.sync_copy(hsb, agg.at[rowv], add=True)

    plsc.subcore_barrier()
    r0 = s * ROWS_PER_SUBCORE
    pltpu.sync_copy(
        agg.at[pl.ds(r0, ROWS_PER_SUBCORE)],
        out_hbm.at[c].at[pl.ds(r0, ROWS_PER_SUBCORE)],
    )


def _sc_edge(hs, ht, ea, row, col):
    mesh = plsc.VectorSubcoreMesh(core_axis_name="c", subcore_axis_name="s")
    run = pl.kernel(
        _sc_edge_body,
        out_type=jax.ShapeDtypeStruct((NC, N_NODES, HIDDEN), jnp.float32),
        mesh=mesh,
        scratch_types=[
            pltpu.VMEM((BLK,), jnp.int32),
            pltpu.VMEM((BLK,), jnp.int32),
            pltpu.VMEM((BLK, HIDDEN), jnp.float32),
            pltpu.VMEM((BLK, HIDDEN), jnp.float32),
            pltpu.VMEM((BLK, HIDDEN), jnp.float32),
            pltpu.VMEM((ZROWS, HIDDEN), jnp.float32),
            pltpu.VMEM_SHARED((N_NODES, HIDDEN), jnp.float32),
            pltpu.SemaphoreType.DMA,
            pltpu.SemaphoreType.DMA,
            pltpu.SemaphoreType.DMA,
        ],
    )
    return run(hs, ht, ea, row, col)


def _node_body(h_ref, aggp_ref, wh_ref, wg_ref, b_ref, o_ref):
    agg = aggp_ref[0] + aggp_ref[1]
    acc = jnp.dot(h_ref[...], wh_ref[...], preferred_element_type=jnp.float32)
    acc = acc + jnp.dot(agg, wg_ref[...], preferred_element_type=jnp.float32)
    o_ref[...] = jnp.maximum(acc + b_ref[...], 0.0)


def _node(h, aggp, wh, wg, b_node):
    return pl.pallas_call(
        _node_body,
        out_shape=jax.ShapeDtypeStruct((N_NODES, HIDDEN), jnp.float32),
    )(h, aggp, wh, wg, b_node.reshape(1, HIDDEN))


def kernel(h, edge_index, edge_attr, W_edge, b_edge, W_node, b_node):
    row = edge_index[0].astype(jnp.int32)
    col = edge_index[1].astype(jnp.int32)
    ws = W_edge[:, :D_FEAT].T
    wt = W_edge[:, D_FEAT:2 * D_FEAT].T
    wa = W_edge[:, 2 * D_FEAT:].T
    wh = W_node[:, :D_FEAT].T
    wg = W_node[:, D_FEAT:].T
    hs, ht = _proj(h, ws, wt)
    ea = _ea(edge_attr, wa, b_edge)
    aggp = _sc_edge(hs, ht, ea, row, col)
    return _node(h, aggp, wh, wg, b_node)


# speedup vs baseline: 3.7897x; 3.7897x over previous
import functools

import jax
import jax.numpy as jnp
from jax import lax
from jax.experimental import pallas as pl
from jax.experimental.pallas import tpu as pltpu
from jax.experimental.pallas import tpu_sc as plsc

N_NODES = 10000
N_EDGES = 320000
D_FEAT = 128
D_EDGE = 16
HIDDEN = 128

NC = 2
NS = 16
LANES = 16
NW = NC * NS
E_PER_W = N_EDGES // NW
BLK = 80
NBLK = E_PER_W // BLK
N_PAD = 10240
ROWS_PER_SUBCORE = N_PAD // NS
ZROWS = 128


def _proj_body(h_ref, ws_ref, wt_ref, hs_ref, ht_ref):
    h = h_ref[...]
    hs_ref[...] = jnp.dot(h, ws_ref[...], preferred_element_type=jnp.float32)
    ht_ref[...] = jnp.dot(h, wt_ref[...], preferred_element_type=jnp.float32)


def _proj(h, ws, wt):
    return pl.pallas_call(
        _proj_body,
        out_shape=[
            jax.ShapeDtypeStruct((N_NODES, HIDDEN), jnp.float32),
            jax.ShapeDtypeStruct((N_NODES, HIDDEN), jnp.float32),
        ],
    )(h, ws, wt)


_EA_BLK = 8000


def _ea_body(a_ref, wa_ref, b_ref, o_ref):
    o_ref[...] = (
        jnp.dot(a_ref[...], wa_ref[...], preferred_element_type=jnp.float32)
        + b_ref[...]
    )


def _ea(edge_attr, wa, b_edge):
    return pl.pallas_call(
        _ea_body,
        grid=(N_EDGES // _EA_BLK,),
        in_specs=[
            pl.BlockSpec((_EA_BLK, D_EDGE), lambda i: (i, 0)),
            pl.BlockSpec((D_EDGE, HIDDEN), lambda i: (0, 0)),
            pl.BlockSpec((1, HIDDEN), lambda i: (0, 0)),
        ],
        out_specs=pl.BlockSpec((_EA_BLK, HIDDEN), lambda i: (i, 0)),
        out_shape=jax.ShapeDtypeStruct((N_EDGES, HIDDEN), jnp.float32),
    )(edge_attr, wa, b_edge.reshape(1, HIDDEN))


def _sc_edge_body(hs_hbm, ht_hbm, ea_hbm, row_hbm, col_hbm, out_hbm,
                  rowv, colv, hsb, htb, eab, zbuf, agg, sem1, sem2, sem3):
    c = lax.axis_index("c")
    s = lax.axis_index("s")
    wid = c * NS + s
    base = wid * E_PER_W

    @pl.loop(0, ZROWS)
    def _zero_rows(i):
        @pl.loop(0, HIDDEN, step=LANES)
        def _zero_lanes(j):
            zbuf[i, pl.ds(j, LANES)] = jnp.zeros((LANES,), jnp.float32)

    @pl.loop(0, ROWS_PER_SUBCORE, step=ZROWS)
    def _zero_copy(r):
        pltpu.sync_copy(zbuf, agg.at[pl.ds(s * ROWS_PER_SUBCORE + r, ZROWS)])

    plsc.subcore_barrier()

    @pl.loop(0, NBLK)
    def _block(b):
        off = base + b * BLK
        pltpu.sync_copy(row_hbm.at[pl.ds(off, BLK)], rowv)
        pltpu.sync_copy(col_hbm.at[pl.ds(off, BLK)], colv)
        ga = pltpu.async_copy(hs_hbm.at[rowv], hsb, sem1)
        gb = pltpu.async_copy(ht_hbm.at[colv], htb, sem2)
        ge = pltpu.async_copy(ea_hbm.at[pl.ds(off, BLK)], eab, sem3)
        ga.wait()
        gb.wait()
        ge.wait()

        @pl.loop(0, BLK)
        def _edge(i):
            @pl.loop(0, HIDDEN, step=LANES)
            def _lanes(j):
                v = (hsb[i, pl.ds(j, LANES)]
                     + htb[i, pl.ds(j, LANES)]
                     + eab[i, pl.ds(j, LANES)])
                hsb[i, pl.ds(j, LANES)] = jnp.maximum(v, 0.0)

        pltpu.sync_copy(hsb, agg.at[rowv], add=True)

    plsc.subcore_barrier()
    r0 = s * ROWS_PER_SUBCORE
    pltpu.sync_copy(
        agg.at[pl.ds(r0, ROWS_PER_SUBCORE)],
        out_hbm.at[c].at[pl.ds(r0, ROWS_PER_SUBCORE)],
    )


def _sc_edge(hs, ht, ea, row, col):
    mesh = plsc.VectorSubcoreMesh(core_axis_name="c", subcore_axis_name="s")
    run = pl.kernel(
        _sc_edge_body,
        out_type=jax.ShapeDtypeStruct((NC, N_PAD, HIDDEN), jnp.float32),
        mesh=mesh,
        scratch_types=[
            pltpu.VMEM((BLK,), jnp.int32),
            pltpu.VMEM((BLK,), jnp.int32),
            pltpu.VMEM((BLK, HIDDEN), jnp.float32),
            pltpu.VMEM((BLK, HIDDEN), jnp.float32),
            pltpu.VMEM((BLK, HIDDEN), jnp.float32),
            pltpu.VMEM((ZROWS, HIDDEN), jnp.float32),
            pltpu.VMEM_SHARED((N_PAD, HIDDEN), jnp.float32),
            pltpu.SemaphoreType.DMA,
            pltpu.SemaphoreType.DMA,
            pltpu.SemaphoreType.DMA,
        ],
    )
    return run(hs, ht, ea, row, col)


def _node_body(h_ref, aggp_ref, wh_ref, wg_ref, b_ref, o_ref):
    agg = aggp_ref[0, :N_NODES, :] + aggp_ref[1, :N_NODES, :]
    acc = jnp.dot(h_ref[...], wh_ref[...], preferred_element_type=jnp.float32)
    acc = acc + jnp.dot(agg, wg_ref[...], preferred_element_type=jnp.float32)
    o_ref[...] = jnp.maximum(acc + b_ref[...], 0.0)


def _node(h, aggp, wh, wg, b_node):
    return pl.pallas_call(
        _node_body,
        out_shape=jax.ShapeDtypeStruct((N_NODES, HIDDEN), jnp.float32),
    )(h, aggp, wh, wg, b_node.reshape(1, HIDDEN))


def kernel(h, edge_index, edge_attr, W_edge, b_edge, W_node, b_node):
    row = edge_index[0].astype(jnp.int32)
    col = edge_index[1].astype(jnp.int32)
    ws = W_edge[:, :D_FEAT].T
    wt = W_edge[:, D_FEAT:2 * D_FEAT].T
    wa = W_edge[:, 2 * D_FEAT:].T
    wh = W_node[:, :D_FEAT].T
    wg = W_node[:, D_FEAT:].T
    hs, ht = _proj(h, ws, wt)
    ea = _ea(edge_attr, wa, b_edge)
    aggp = _sc_edge(hs, ht, ea, row, col)
    return _node(h, aggp, wh, wg, b_node)
